# baseline (speedup 1.0000x reference)
import functools

import jax
import jax.numpy as jnp
from jax import lax
from jax.experimental import pallas as pl
from jax.experimental.pallas import tpu as pltpu
from jax.experimental.pallas import tpu_sc as plsc

_B = 16384
_D = 64
_NC = 2
_NS = 16
_L = 16
_NW = _NC * _NS
_BPW = _B // _NW
_C = 128
_NCHUNK = _BPW // _C

_INV_SQRT_2PI = 0.3989422804014327


def _body(x_hbm, idx_hbm, loc_hbm, pi_hbm, spike_hbm, slab_hbm, out_hbm,
          idx_v0, idx_v1, x_v, loc_v, pi_v, spike_v, slab_v, out_v, sems):
    wid = lax.axis_index("s") * _NC + lax.axis_index("c")
    wbase = wid * _BPW
    idx_bufs = (idx_v0, idx_v1)

    def start_chunk(c, buf):
        base = wbase + c * _C
        idx_v = idx_bufs[buf]
        pltpu.sync_copy(idx_hbm.at[pl.ds(base, _C)], idx_v)
        gl = pltpu.async_copy(loc_hbm.at[idx_v], loc_v.at[buf], sems.at[buf, 0])
        gp = pltpu.async_copy(pi_hbm.at[idx_v], pi_v.at[buf], sems.at[buf, 1])
        gs = pltpu.async_copy(spike_hbm.at[idx_v], spike_v.at[buf], sems.at[buf, 2])
        gb = pltpu.async_copy(slab_hbm.at[idx_v], slab_v.at[buf], sems.at[buf, 3])
        pltpu.sync_copy(x_hbm.at[pl.ds(base, _C)], x_v.at[buf])
        return gl, gp, gs, gb

    def compute_chunk(c, buf, dmas):
        for d in dmas:
            d.wait()

        def row(r, _):
            for j in range(_D // _L):
                sl = pl.ds(j * _L, _L)
                x = x_v[buf, r, sl]
                lo = loc_v[buf, r, sl]
                p = pi_v[buf, r, sl]
                sp = spike_v[buf, r, sl]
                sb = slab_v[buf, r, sl]
                diff = x - lo
                isp = 1.0 / sp
                isb = 1.0 / sb
                zs = diff * isp
                zb = diff * isb
                es = jnp.exp(-0.5 * (zs * zs))
                eb = jnp.exp(-0.5 * (zb * zb))
                spike_term = (p * _INV_SQRT_2PI) * isp * es
                slab_term = ((1.0 - p) * _INV_SQRT_2PI) * isb * eb
                out_v[buf, r, sl] = spike_term + slab_term
            return 0

        lax.fori_loop(0, _C, row, 0)
        pltpu.sync_copy(out_v.at[buf], out_hbm.at[pl.ds(wbase + c * _C, _C)])

    dmas = start_chunk(0, 0)
    for c in range(_NCHUNK):
        nxt = None
        if c + 1 < _NCHUNK:
            nxt = start_chunk(c + 1, (c + 1) % 2)
        compute_chunk(c, c % 2, dmas)
        dmas = nxt


@functools.partial(jax.jit)
def _spike_slab_sc(X, indices, loc, pi, spike, slab):
    mesh = plsc.VectorSubcoreMesh(core_axis_name="c", subcore_axis_name="s",
                                  num_cores=_NC, num_subcores=_NS)
    kern = pl.kernel(
        _body,
        out_type=jax.ShapeDtypeStruct((_B, _D), jnp.float32),
        mesh=mesh,
        scratch_types=[
            pltpu.VMEM((_C,), jnp.int32),
            pltpu.VMEM((_C,), jnp.int32),
            pltpu.VMEM((2, _C, _D), jnp.float32),
            pltpu.VMEM((2, _C, _D), jnp.float32),
            pltpu.VMEM((2, _C, _D), jnp.float32),
            pltpu.VMEM((2, _C, _D), jnp.float32),
            pltpu.VMEM((2, _C, _D), jnp.float32),
            pltpu.VMEM((2, _C, _D), jnp.float32),
            pltpu.SemaphoreType.DMA((2, 4)),
        ],
        compiler_params=pltpu.CompilerParams(use_tc_tiling_on_sc=False),
    )
    return kern(X, indices, loc, pi, spike, slab)


def kernel(X, indices, loc, pi, spike, slab):
    return _spike_slab_sc(X, indices.astype(jnp.int32), loc, pi, spike, slab)
